# baseline (speedup 1.0000x reference)
import jax
import jax.numpy as jnp
from jax import lax
from jax.experimental import pallas as pl
from jax.experimental.pallas import tpu as pltpu
from jax.experimental.pallas import tpu_sc as plsc

_TOPK = 512
_INT_MIN = -(2**31)
_N = 32768
_B = 1024
_NW = 32
_RPW = _B // _NW


def _i32(v):
    return jnp.int32(v)


def _sc_body(x_hbm, o_hbm, buf, hist, totbuf, cumbuf, pbuf):
    cid = lax.axis_index("c")
    sid = lax.axis_index("s")
    wid = sid * 2 + cid
    lane = lax.iota(jnp.int32, 16)
    zeros16 = jnp.zeros((16,), jnp.int32)
    ones16 = jnp.ones((16,), jnp.int32)

    def zero_hist(i, carry):
        hist[pl.ds(i * 16, 16)] = zeros16
        return carry

    lax.fori_loop(0, _N // 16, zero_hist, 0)

    def scan_level(nb, n_l, k_l):
        cchunks = nb // 16

        def phase_a(ci, carry):
            acc = zeros16
            for l in range(16):
                off = l * nb + ci * 16
                acc = acc + hist[pl.ds(off, 16)]
                hist[pl.ds(off, 16)] = zeros16
            totbuf[pl.ds(ci * 16, 16)] = acc
            cumbuf[pl.ds(ci * 16, 16)] = lax.cumsum(acc, axis=0)
            return carry

        lax.fori_loop(0, cchunks, phase_a, 0)

        groups = cchunks // 16
        prev = zeros16
        for g in range(groups):
            idx = lane * 16 + 15 + g * 256
            ct = plsc.load_gather(cumbuf, [idx])
            cum_ct = lax.cumsum(ct, axis=0)
            pbuf[pl.ds(g * 16, 16)] = cum_ct - ct + prev
            prev = prev + jnp.max(cum_ct, axis=0)

        def phase_c(ci, cnt):
            t = totbuf[pl.ds(ci * 16, 16)]
            pfx = plsc.load_gather(pbuf, [jnp.broadcast_to(ci, (16,)).astype(jnp.int32)])
            cm = cumbuf[pl.ds(ci * 16, 16)] + pfx
            cond = (n_l - cm + t) >= k_l
            return cnt + plsc.all_reduce_population_count(cond)

        cnt = lax.fori_loop(0, cchunks, phase_c, zeros16)
        bstar = cnt - 1
        cumb = plsc.load_gather(cumbuf, [bstar]) + plsc.load_gather(
            pbuf, [lax.shift_right_logical(bstar, 4)]
        )
        totb = plsc.load_gather(totbuf, [bstar])
        return bstar, cumb, totb

    base2048 = lane * 2048
    base1024 = lane * 1024

    def process_row(row):
        pltpu.sync_copy(x_hbm.at[row], buf)

        def p1(i, carry):
            x = buf[pl.ds(i * 16, 16)]
            bits = lax.bitcast_convert_type(x, jnp.int32)
            u = jnp.where(bits < 0, bits ^ _i32(-1), bits ^ _i32(_INT_MIN))
            buf[pl.ds(i * 16, 16)] = lax.bitcast_convert_type(u, jnp.float32)
            b1v = lax.shift_right_logical(u, 21)
            plsc.addupdate_scatter(hist, [base2048 + b1v], ones16)
            return carry

        lax.fori_loop(0, _N // 16, p1, 0)
        n1 = jnp.broadcast_to(_i32(_N), (16,))
        k1 = jnp.broadcast_to(_i32(_TOPK), (16,))
        b1, cumb1, totb1 = scan_level(2048, n1, k1)
        n2 = totb1
        k2 = k1 - (n1 - cumb1)

        def p2(i, carry):
            u = lax.bitcast_convert_type(buf[pl.ds(i * 16, 16)], jnp.int32)
            m = lax.shift_right_logical(u, 21) == b1
            b2v = lax.shift_right_logical(u, 10) & _i32(0x7FF)
            plsc.addupdate_scatter(hist, [base2048 + b2v], ones16, mask=m)
            return carry

        lax.fori_loop(0, _N // 16, p2, 0)
        b2, cumb2, totb2 = scan_level(2048, n2, k2)
        n3 = totb2
        k3 = k2 - (n2 - cumb2)
        pref22 = (b1 << 11) | b2

        def p3(i, carry):
            u = lax.bitcast_convert_type(buf[pl.ds(i * 16, 16)], jnp.int32)
            m = lax.shift_right_logical(u, 10) == pref22
            b3v = u & _i32(0x3FF)
            plsc.addupdate_scatter(hist, [base1024 + b3v], ones16, mask=m)
            return carry

        lax.fori_loop(0, _N // 16, p3, 0)
        b3, _, _ = scan_level(1024, n3, k3)

        kth_u = (b1 << 21) | (b2 << 10) | b3
        sthr = jnp.maximum(kth_u ^ _i32(_INT_MIN), 0)

        def p4(i, carry):
            u = lax.bitcast_convert_type(buf[pl.ds(i * 16, 16)], jnp.int32)
            s = u ^ _i32(_INT_MIN)
            o = jnp.where(s >= sthr, s, 0)
            buf[pl.ds(i * 16, 16)] = lax.bitcast_convert_type(o, jnp.float32)
            return carry

        lax.fori_loop(0, _N // 16, p4, 0)
        pltpu.sync_copy(buf, o_hbm.at[row])

    def row_loop(r, carry):
        process_row(wid * _RPW + r)
        return carry

    lax.fori_loop(0, _RPW, row_loop, 0)


def kernel(features):
    b, l, d = features.shape
    flat = features.reshape(b, l * d)
    mesh = plsc.VectorSubcoreMesh(core_axis_name="c", subcore_axis_name="s")
    out = pl.kernel(
        _sc_body,
        out_type=jax.ShapeDtypeStruct((b, l * d), jnp.float32),
        mesh=mesh,
        compiler_params=pltpu.CompilerParams(needs_layout_passes=False),
        scratch_types=[
            pltpu.VMEM((_N,), jnp.float32),
            pltpu.VMEM((_N,), jnp.int32),
            pltpu.VMEM((2048,), jnp.int32),
            pltpu.VMEM((2048,), jnp.int32),
            pltpu.VMEM((128,), jnp.int32),
        ],
    )(flat)
    return out.reshape(b, l, d)


# speedup vs baseline: 1.2248x; 1.2248x over previous
import jax
import jax.numpy as jnp
from jax import lax
from jax.experimental import pallas as pl
from jax.experimental.pallas import tpu as pltpu
from jax.experimental.pallas import tpu_sc as plsc

_TOPK = 512
_INT_MIN = -(2**31)
_N = 32768
_B = 1024
_NW = 32
_RPW = _B // _NW


def _i32(v):
    return jnp.int32(v)


def _sc_body(x_hbm, o_hbm, buf, hist, totbuf, cumbuf, pbuf):
    cid = lax.axis_index("c")
    sid = lax.axis_index("s")
    wid = sid * 2 + cid
    lane = lax.iota(jnp.int32, 16)
    zeros16 = jnp.zeros((16,), jnp.int32)
    ones16 = jnp.ones((16,), jnp.int32)

    def zero_hist(i, carry):
        hist[pl.ds(i * 16, 16)] = zeros16
        return carry

    lax.fori_loop(0, _N // 16, zero_hist, 0, unroll=8)

    def scan_level(nb, n_l, k_l):
        cchunks = nb // 16

        def phase_a(ci, carry):
            acc = zeros16
            for l in range(16):
                off = l * nb + ci * 16
                acc = acc + hist[pl.ds(off, 16)]
                hist[pl.ds(off, 16)] = zeros16
            totbuf[pl.ds(ci * 16, 16)] = acc
            cumbuf[pl.ds(ci * 16, 16)] = lax.cumsum(acc, axis=0)
            return carry

        lax.fori_loop(0, cchunks, phase_a, 0, unroll=2)

        groups = cchunks // 16
        prev = zeros16
        for g in range(groups):
            idx = lane * 16 + 15 + g * 256
            ct = plsc.load_gather(cumbuf, [idx])
            cum_ct = lax.cumsum(ct, axis=0)
            pbuf[pl.ds(g * 16, 16)] = cum_ct - ct + prev
            prev = prev + jnp.max(cum_ct, axis=0)

        def phase_c(ci, cnt):
            t = totbuf[pl.ds(ci * 16, 16)]
            pfx = plsc.load_gather(pbuf, [jnp.broadcast_to(ci, (16,)).astype(jnp.int32)])
            cm = cumbuf[pl.ds(ci * 16, 16)] + pfx
            cond = (n_l - cm + t) >= k_l
            return cnt + plsc.all_reduce_population_count(cond)

        cnt = lax.fori_loop(0, cchunks, phase_c, zeros16, unroll=4)
        bstar = cnt - 1
        cumb = plsc.load_gather(cumbuf, [bstar]) + plsc.load_gather(
            pbuf, [lax.shift_right_logical(bstar, 4)]
        )
        totb = plsc.load_gather(totbuf, [bstar])
        return bstar, cumb, totb

    base2048 = lane * 2048
    base1024 = lane * 1024

    def process_row(row):
        pltpu.sync_copy(x_hbm.at[row], buf)

        def p1(i, carry):
            x = buf[pl.ds(i * 16, 16)]
            bits = lax.bitcast_convert_type(x, jnp.int32)
            u = jnp.where(bits < 0, bits ^ _i32(-1), bits ^ _i32(_INT_MIN))
            buf[pl.ds(i * 16, 16)] = lax.bitcast_convert_type(u, jnp.float32)
            b1v = lax.shift_right_logical(u, 21)
            plsc.addupdate_scatter(hist, [base2048 + b1v], ones16)
            return carry

        lax.fori_loop(0, _N // 16, p1, 0, unroll=8)
        n1 = jnp.broadcast_to(_i32(_N), (16,))
        k1 = jnp.broadcast_to(_i32(_TOPK), (16,))
        b1, cumb1, totb1 = scan_level(2048, n1, k1)
        n2 = totb1
        k2 = k1 - (n1 - cumb1)

        def p2(i, carry):
            u = lax.bitcast_convert_type(buf[pl.ds(i * 16, 16)], jnp.int32)
            m = lax.shift_right_logical(u, 21) == b1
            b2v = lax.shift_right_logical(u, 10) & _i32(0x7FF)
            plsc.addupdate_scatter(hist, [base2048 + b2v], ones16, mask=m)
            return carry

        lax.fori_loop(0, _N // 16, p2, 0, unroll=8)
        b2, cumb2, totb2 = scan_level(2048, n2, k2)
        n3 = totb2
        k3 = k2 - (n2 - cumb2)
        pref22 = (b1 << 11) | b2

        def p3(i, carry):
            u = lax.bitcast_convert_type(buf[pl.ds(i * 16, 16)], jnp.int32)
            m = lax.shift_right_logical(u, 10) == pref22
            b3v = u & _i32(0x3FF)
            plsc.addupdate_scatter(hist, [base1024 + b3v], ones16, mask=m)
            return carry

        lax.fori_loop(0, _N // 16, p3, 0, unroll=8)
        b3, _, _ = scan_level(1024, n3, k3)

        kth_u = (b1 << 21) | (b2 << 10) | b3
        sthr = jnp.maximum(kth_u ^ _i32(_INT_MIN), 0)

        def p4(i, carry):
            u = lax.bitcast_convert_type(buf[pl.ds(i * 16, 16)], jnp.int32)
            s = u ^ _i32(_INT_MIN)
            o = jnp.where(s >= sthr, s, 0)
            buf[pl.ds(i * 16, 16)] = lax.bitcast_convert_type(o, jnp.float32)
            return carry

        lax.fori_loop(0, _N // 16, p4, 0, unroll=8)
        pltpu.sync_copy(buf, o_hbm.at[row])

    def row_loop(r, carry):
        process_row(wid * _RPW + r)
        return carry

    lax.fori_loop(0, _RPW, row_loop, 0)


def kernel(features):
    b, l, d = features.shape
    flat = features.reshape(b, l * d)
    mesh = plsc.VectorSubcoreMesh(core_axis_name="c", subcore_axis_name="s")
    out = pl.kernel(
        _sc_body,
        out_type=jax.ShapeDtypeStruct((b, l * d), jnp.float32),
        mesh=mesh,
        compiler_params=pltpu.CompilerParams(needs_layout_passes=False),
        scratch_types=[
            pltpu.VMEM((_N,), jnp.float32),
            pltpu.VMEM((_N,), jnp.int32),
            pltpu.VMEM((2048,), jnp.int32),
            pltpu.VMEM((2048,), jnp.int32),
            pltpu.VMEM((128,), jnp.int32),
        ],
    )(flat)
    return out.reshape(b, l, d)


# speedup vs baseline: 3.5991x; 2.9386x over previous
import jax
import jax.numpy as jnp
from jax import lax
from jax.experimental import pallas as pl
from jax.experimental.pallas import tpu as pltpu
from jax.experimental.pallas import tpu_sc as plsc

_TOPK = 512
_INT_MIN = -(2**31)
_N = 32768
_B = 1024
_NW = 32
_RPW = _B // _NW


def _i32(v):
    return jnp.int32(v)


def _sc_body(x_hbm, o_hbm, buf, hist, totbuf, cumbuf, pbuf):
    cid = lax.axis_index("c")
    sid = lax.axis_index("s")
    wid = sid * 2 + cid
    lane = lax.iota(jnp.int32, 16)
    zeros16 = jnp.zeros((16,), jnp.int32)
    ones16 = jnp.ones((16,), jnp.int32)

    @plsc.parallel_loop(0, _N // 16, unroll=8)
    def _(i):
        hist[pl.ds(i * 16, 16)] = zeros16

    def scan_level(nb, n_l, k_l):
        cchunks = nb // 16

        @plsc.parallel_loop(0, cchunks, unroll=2)
        def _(ci):
            acc = zeros16
            for l in range(16):
                off = l * nb + ci * 16
                acc = acc + hist[pl.ds(off, 16)]
                hist[pl.ds(off, 16)] = zeros16
            totbuf[pl.ds(ci * 16, 16)] = acc
            cumbuf[pl.ds(ci * 16, 16)] = lax.cumsum(acc, axis=0)

        groups = cchunks // 16
        prev = zeros16
        for g in range(groups):
            idx = lane * 16 + 15 + g * 256
            ct = plsc.load_gather(cumbuf, [idx])
            cum_ct = lax.cumsum(ct, axis=0)
            pbuf[pl.ds(g * 16, 16)] = cum_ct - ct + prev
            prev = prev + jnp.max(cum_ct, axis=0)

        @plsc.parallel_loop(0, cchunks, unroll=4, carry=zeros16)
        def cnt(ci, acc):
            t = totbuf[pl.ds(ci * 16, 16)]
            pfx = plsc.load_gather(pbuf, [jnp.broadcast_to(ci, (16,)).astype(jnp.int32)])
            cm = cumbuf[pl.ds(ci * 16, 16)] + pfx
            cond = (n_l - cm + t) >= k_l
            return acc + plsc.all_reduce_population_count(cond)

        bstar = cnt - 1
        cumb = plsc.load_gather(cumbuf, [bstar]) + plsc.load_gather(
            pbuf, [lax.shift_right_logical(bstar, 4)]
        )
        totb = plsc.load_gather(totbuf, [bstar])
        return bstar, cumb, totb

    base2048 = lane * 2048
    base1024 = lane * 1024

    def process_row(row):
        pltpu.sync_copy(x_hbm.at[row], buf)

        @plsc.parallel_loop(0, _N // 16, unroll=8)
        def _(i):
            x = buf[pl.ds(i * 16, 16)]
            bits = lax.bitcast_convert_type(x, jnp.int32)
            u = jnp.where(bits < 0, bits ^ _i32(-1), bits ^ _i32(_INT_MIN))
            buf[pl.ds(i * 16, 16)] = lax.bitcast_convert_type(u, jnp.float32)
            b1v = lax.shift_right_logical(u, 21)
            plsc.addupdate_scatter(hist, [base2048 + b1v], ones16)

        n1 = jnp.broadcast_to(_i32(_N), (16,))
        k1 = jnp.broadcast_to(_i32(_TOPK), (16,))
        b1, cumb1, totb1 = scan_level(2048, n1, k1)
        n2 = totb1
        k2 = k1 - (n1 - cumb1)

        @plsc.parallel_loop(0, _N // 16, unroll=8)
        def _(i):
            u = lax.bitcast_convert_type(buf[pl.ds(i * 16, 16)], jnp.int32)
            m = lax.shift_right_logical(u, 21) == b1
            b2v = lax.shift_right_logical(u, 10) & _i32(0x7FF)
            plsc.addupdate_scatter(hist, [base2048 + b2v], ones16, mask=m)

        b2, cumb2, totb2 = scan_level(2048, n2, k2)
        n3 = totb2
        k3 = k2 - (n2 - cumb2)
        pref22 = (b1 << 11) | b2

        @plsc.parallel_loop(0, _N // 16, unroll=8)
        def _(i):
            u = lax.bitcast_convert_type(buf[pl.ds(i * 16, 16)], jnp.int32)
            m = lax.shift_right_logical(u, 10) == pref22
            b3v = u & _i32(0x3FF)
            plsc.addupdate_scatter(hist, [base1024 + b3v], ones16, mask=m)

        b3, _, _ = scan_level(1024, n3, k3)

        kth_u = (b1 << 21) | (b2 << 10) | b3
        sthr = jnp.maximum(kth_u ^ _i32(_INT_MIN), 0)

        @plsc.parallel_loop(0, _N // 16, unroll=8)
        def _(i):
            u = lax.bitcast_convert_type(buf[pl.ds(i * 16, 16)], jnp.int32)
            s = u ^ _i32(_INT_MIN)
            o = jnp.where(s >= sthr, s, 0)
            buf[pl.ds(i * 16, 16)] = lax.bitcast_convert_type(o, jnp.float32)

        pltpu.sync_copy(buf, o_hbm.at[row])

    def row_loop(r, carry):
        process_row(wid * _RPW + r)
        return carry

    lax.fori_loop(0, _RPW, row_loop, 0)


def kernel(features):
    b, l, d = features.shape
    flat = features.reshape(b, l * d)
    mesh = plsc.VectorSubcoreMesh(core_axis_name="c", subcore_axis_name="s")
    out = pl.kernel(
        _sc_body,
        out_type=jax.ShapeDtypeStruct((b, l * d), jnp.float32),
        mesh=mesh,
        compiler_params=pltpu.CompilerParams(needs_layout_passes=False),
        scratch_types=[
            pltpu.VMEM((_N,), jnp.float32),
            pltpu.VMEM((_N,), jnp.int32),
            pltpu.VMEM((2048,), jnp.int32),
            pltpu.VMEM((2048,), jnp.int32),
            pltpu.VMEM((128,), jnp.int32),
        ],
    )(flat)
    return out.reshape(b, l, d)
